# baseline (speedup 1.0000x reference)
import functools

import jax
import jax.numpy as jnp
from jax import lax
from jax.experimental import pallas as pl
from jax.experimental.pallas import tpu as pltpu
from jax.experimental.pallas import tpu_sc as plsc

_GRID_SIZE = 128
_MIDPOINT = 128
_B = 1048576
_R = 1024
_C = 1024
_ROWS_PER_STEP = 128

_NC = 2
_NS = 16
_NW = _NC * _NS
_BPW = _B // _NW


def _spread_bits(x):
    x = x & jnp.uint32(0x3FF)
    x = (x | (x << 16)) & jnp.uint32(0x030000FF)
    x = (x | (x << 8)) & jnp.uint32(0x0300F00F)
    x = (x | (x << 4)) & jnp.uint32(0x030C30C3)
    x = (x | (x << 2)) & jnp.uint32(0x09249249)
    return x


def _index_body(o_ref, v_ref, out_ref):
    vx = v_ref[0]
    vy = v_ref[1]
    vz = v_ref[2]
    denom = jnp.maximum(jnp.maximum(jnp.abs(vx), jnp.abs(vy)), jnp.abs(vz))
    a = vx / denom
    b = vy / denom
    c = vz / denom
    one = jnp.float32(1.0)
    conds = (a >= one, a <= -one, b >= one, b <= -one, c >= one, c <= -one)
    face = jnp.zeros(a.shape, jnp.int32)
    for i, cond in enumerate(conds):
        face = jnp.where(cond, jnp.int32(i), face)

    def cell(p):
        q = jnp.clip((p / 2.0 + 0.5) * _GRID_SIZE, 0.0, float(_GRID_SIZE - 1))
        return _spread_bits(jnp.floor(q).astype(jnp.int32).astype(jnp.uint32))

    morton = (
        cell(o_ref[0]) | (cell(o_ref[1]) << 1) | (cell(o_ref[2]) << 2)
    ).astype(jnp.int32)
    out_ref[...] = morton * 6 + face


def _compute_flat_indices(o3, v3):
    return pl.pallas_call(
        _index_body,
        grid=(_R // _ROWS_PER_STEP,),
        in_specs=[
            pl.BlockSpec((3, _ROWS_PER_STEP, _C), lambda i: (0, i, 0)),
            pl.BlockSpec((3, _ROWS_PER_STEP, _C), lambda i: (0, i, 0)),
        ],
        out_specs=pl.BlockSpec((_ROWS_PER_STEP, _C), lambda i: (i, 0)),
        out_shape=jax.ShapeDtypeStruct((_R, _C), jnp.int32),
    )(o3, v3)


@functools.cache
def _make_gather_compare():
    @functools.partial(
        pl.kernel,
        mesh=plsc.VectorSubcoreMesh(core_axis_name="c", subcore_axis_name="s"),
        out_type=jax.ShapeDtypeStruct((_B,), jnp.int32),
        scratch_types=[
            pltpu.VMEM((_BPW,), jnp.int32),
            pltpu.VMEM((_BPW,), jnp.int32),
            pltpu.SemaphoreType.DMA,
        ],
    )
    def _gather_compare(idx_hbm, table_hbm, out_hbm, idx_v, vals_v, sem):
        wid = lax.axis_index("s") * _NC + lax.axis_index("c")
        base = wid * _BPW
        pltpu.sync_copy(idx_hbm.at[pl.ds(base, _BPW)], idx_v)
        pltpu.async_copy(table_hbm.at[idx_v], vals_v, sem).wait()

        thresh = jnp.full((16,), _MIDPOINT, jnp.int32)
        ones = jnp.full((16,), 1, jnp.int32)
        zeros = jnp.full((16,), 0, jnp.int32)

        def body(i, carry):
            sl = pl.ds(i * 16, 16)
            vals_v[sl] = jnp.where(vals_v[sl] > thresh, ones, zeros)
            return carry

        lax.fori_loop(0, _BPW // 16, body, 0)
        pltpu.sync_copy(vals_v, out_hbm.at[pl.ds(base, _BPW)])

    return _gather_compare


def kernel(norm_ray_origins, viewdirs, cache):
    o3 = norm_ray_origins.T.reshape(3, _R, _C)
    v3 = viewdirs.T.reshape(3, _R, _C)
    flat_idx = _compute_flat_indices(o3, v3).reshape(_B)
    vals01 = _make_gather_compare()(flat_idx, cache.reshape(-1))
    return vals01.astype(jnp.bool_)


# speedup vs baseline: 6.4483x; 6.4483x over previous
import functools

import jax
import jax.numpy as jnp
from jax import lax
from jax.experimental import pallas as pl
from jax.experimental.pallas import tpu as pltpu
from jax.experimental.pallas import tpu_sc as plsc

_GRID_SIZE = 128
_MIDPOINT = 128
_B = 1048576
_NCELL = _GRID_SIZE ** 3
_R = 1024
_C = 1024
_ROWS_PER_STEP = 128
_PACK_COLS = 131072

_NC = 2
_NS = 16
_NW = _NC * _NS
_BPW = _B // _NW


def _spread_bits(x):
    x = x & jnp.uint32(0x3FF)
    x = (x | (x << 16)) & jnp.uint32(0x030000FF)
    x = (x | (x << 8)) & jnp.uint32(0x0300F00F)
    x = (x | (x << 4)) & jnp.uint32(0x030C30C3)
    x = (x | (x << 2)) & jnp.uint32(0x09249249)
    return x


def _pack_body(ct_ref, out_ref):
    blk = ct_ref[...]
    f = lax.broadcasted_iota(jnp.int32, blk.shape, 0)
    bits = jnp.where(blk > _MIDPOINT, jnp.int32(1) << f, jnp.int32(0))
    out_ref[...] = jnp.sum(bits, axis=0, keepdims=True)[None]


def _pack_table(cache_t):
    grid = _NCELL // _PACK_COLS
    return pl.pallas_call(
        _pack_body,
        grid=(grid,),
        in_specs=[pl.BlockSpec((6, _PACK_COLS), lambda i: (0, i))],
        out_specs=pl.BlockSpec((1, 1, _PACK_COLS), lambda i: (i, 0, 0)),
        out_shape=jax.ShapeDtypeStruct((grid, 1, _PACK_COLS), jnp.int32),
    )(cache_t)


def _index_body(o_ref, v_ref, cell_ref, mask_ref):
    vx = v_ref[0]
    vy = v_ref[1]
    vz = v_ref[2]
    denom = jnp.maximum(jnp.maximum(jnp.abs(vx), jnp.abs(vy)), jnp.abs(vz))
    a = vx / denom
    b = vy / denom
    c = vz / denom
    one = jnp.float32(1.0)
    conds = (a >= one, a <= -one, b >= one, b <= -one, c >= one, c <= -one)
    face = jnp.zeros(vx.shape, jnp.int32)
    for i, cond in enumerate(conds):
        face = jnp.where(cond, jnp.int32(i), face)

    def cell(p):
        q = jnp.clip((p / 2.0 + 0.5) * _GRID_SIZE, 0.0, float(_GRID_SIZE - 1))
        return _spread_bits(jnp.floor(q).astype(jnp.int32).astype(jnp.uint32))

    morton = cell(o_ref[0]) | (cell(o_ref[1]) << 1) | (cell(o_ref[2]) << 2)
    cell_ref[...] = morton.astype(jnp.int32)
    mask_ref[...] = jnp.int32(1) << face


def _compute_indices(o3, v3):
    return pl.pallas_call(
        _index_body,
        grid=(_R // _ROWS_PER_STEP,),
        in_specs=[
            pl.BlockSpec((3, _ROWS_PER_STEP, _C), lambda i: (0, i, 0)),
            pl.BlockSpec((3, _ROWS_PER_STEP, _C), lambda i: (0, i, 0)),
        ],
        out_specs=[
            pl.BlockSpec((_ROWS_PER_STEP, _C), lambda i: (i, 0)),
            pl.BlockSpec((_ROWS_PER_STEP, _C), lambda i: (i, 0)),
        ],
        out_shape=[
            jax.ShapeDtypeStruct((_R, _C), jnp.int32),
            jax.ShapeDtypeStruct((_R, _C), jnp.int32),
        ],
    )(o3, v3)


@functools.cache
def _make_gather():
    @functools.partial(
        pl.kernel,
        mesh=plsc.VectorSubcoreMesh(core_axis_name="c", subcore_axis_name="s"),
        out_type=jax.ShapeDtypeStruct((_B,), jnp.int32),
        scratch_types=[
            pltpu.VMEM((_BPW,), jnp.int32),
            pltpu.VMEM((_BPW,), jnp.int32),
            pltpu.VMEM((_BPW,), jnp.int32),
            pltpu.SemaphoreType.DMA,
        ],
    )
    def _gather(cell_hbm, mask_hbm, table_hbm, out_hbm, idx_v, mask_v,
                words_v, sem):
        wid = lax.axis_index("s") * _NC + lax.axis_index("c")
        base = wid * _BPW
        pltpu.sync_copy(cell_hbm.at[pl.ds(base, _BPW)], idx_v)
        pltpu.sync_copy(mask_hbm.at[pl.ds(base, _BPW)], mask_v)
        pltpu.async_copy(table_hbm.at[idx_v], words_v, sem).wait()

        def body(i, carry):
            sl = pl.ds(i * 16, 16)
            words_v[sl] = words_v[sl] & mask_v[sl]
            return carry

        lax.fori_loop(0, _BPW // 16, body, 0)
        pltpu.sync_copy(words_v, out_hbm.at[pl.ds(base, _BPW)])

    return _gather


def kernel(norm_ray_origins, viewdirs, cache):
    o3 = norm_ray_origins.T.reshape(3, _R, _C)
    v3 = viewdirs.T.reshape(3, _R, _C)
    cell, mask = _compute_indices(o3, v3)
    table = _pack_table(cache.T).reshape(_NCELL)
    vals = _make_gather()(cell.reshape(_B), mask.reshape(_B), table)
    return vals.astype(jnp.bool_)


# speedup vs baseline: 6.7586x; 1.0481x over previous
import functools

import jax
import jax.numpy as jnp
from jax import lax
from jax.experimental import pallas as pl
from jax.experimental.pallas import tpu as pltpu
from jax.experimental.pallas import tpu_sc as plsc

_GRID_SIZE = 128
_MIDPOINT = 128
_B = 1048576
_NCELL = _GRID_SIZE ** 3
_NWORD = _NCELL // 4
_R = 1024
_C = 1024
_ROWS_PER_STEP = 128
_PACK_COLS = 131072

_NC = 2
_NS = 16
_NW = _NC * _NS
_BPW = _B // _NW
_CHUNK = 4096
_NCHUNK = _BPW // _CHUNK


def _spread_bits(x):
    x = x & jnp.uint32(0x3FF)
    x = (x | (x << 16)) & jnp.uint32(0x030000FF)
    x = (x | (x << 8)) & jnp.uint32(0x0300F00F)
    x = (x | (x << 4)) & jnp.uint32(0x030C30C3)
    x = (x | (x << 2)) & jnp.uint32(0x09249249)
    return x


def _pack_body(c0_ref, c1_ref, c2_ref, c3_ref, out_ref):
    def byte(ref):
        blk = ref[...]
        f = lax.broadcasted_iota(jnp.int32, blk.shape, 0)
        bits = jnp.where(blk > _MIDPOINT, jnp.int32(1) << f, jnp.int32(0))
        return jnp.sum(bits, axis=0, keepdims=True)

    word = (byte(c0_ref) | (byte(c1_ref) << 8) | (byte(c2_ref) << 16)
            | (byte(c3_ref) << 24))
    out_ref[...] = word[None]


def _pack_table(cache_t):
    grid = _NWORD // _PACK_COLS
    blocks_per_byte = _NWORD // _PACK_COLS

    def spec(k):
        return pl.BlockSpec((6, _PACK_COLS),
                            lambda j, k=k: (0, j + blocks_per_byte * k))

    return pl.pallas_call(
        _pack_body,
        grid=(grid,),
        in_specs=[spec(0), spec(1), spec(2), spec(3)],
        out_specs=pl.BlockSpec((1, 1, _PACK_COLS), lambda j: (j, 0, 0)),
        out_shape=jax.ShapeDtypeStruct((grid, 1, _PACK_COLS), jnp.int32),
    )(cache_t, cache_t, cache_t, cache_t)


def _index_body(o_ref, v_ref, cell_ref, mask_ref):
    vx = v_ref[0]
    vy = v_ref[1]
    vz = v_ref[2]
    denom = jnp.maximum(jnp.maximum(jnp.abs(vx), jnp.abs(vy)), jnp.abs(vz))
    a = vx / denom
    b = vy / denom
    c = vz / denom
    one = jnp.float32(1.0)
    conds = (a >= one, a <= -one, b >= one, b <= -one, c >= one, c <= -one)
    face = jnp.zeros(vx.shape, jnp.int32)
    for i, cond in enumerate(conds):
        face = jnp.where(cond, jnp.int32(i), face)

    def cell(p):
        q = jnp.clip((p / 2.0 + 0.5) * _GRID_SIZE, 0.0, float(_GRID_SIZE - 1))
        return _spread_bits(jnp.floor(q).astype(jnp.int32).astype(jnp.uint32))

    morton = (cell(o_ref[0]) | (cell(o_ref[1]) << 1)
              | (cell(o_ref[2]) << 2)).astype(jnp.int32)
    cell_ref[...] = morton & jnp.int32(_NWORD - 1)
    mask_ref[...] = jnp.int32(1) << (face + ((morton >> 19) << 3))


def _compute_indices(o3, v3):
    return pl.pallas_call(
        _index_body,
        grid=(_R // _ROWS_PER_STEP,),
        in_specs=[
            pl.BlockSpec((3, _ROWS_PER_STEP, _C), lambda i: (0, i, 0)),
            pl.BlockSpec((3, _ROWS_PER_STEP, _C), lambda i: (0, i, 0)),
        ],
        out_specs=[
            pl.BlockSpec((_ROWS_PER_STEP, _C), lambda i: (i, 0)),
            pl.BlockSpec((_ROWS_PER_STEP, _C), lambda i: (i, 0)),
        ],
        out_shape=[
            jax.ShapeDtypeStruct((_R, _C), jnp.int32),
            jax.ShapeDtypeStruct((_R, _C), jnp.int32),
        ],
    )(o3, v3)


@functools.cache
def _make_gather():
    @functools.partial(
        pl.kernel,
        mesh=plsc.VectorSubcoreMesh(core_axis_name="c", subcore_axis_name="s"),
        out_type=jax.ShapeDtypeStruct((_B,), jnp.int32),
        scratch_types=[
            pltpu.VMEM((_BPW,), jnp.int32),
            pltpu.VMEM((_BPW,), jnp.int32),
            pltpu.VMEM((_BPW,), jnp.int32),
            pltpu.SemaphoreType.DMA,
            pltpu.SemaphoreType.DMA,
            pltpu.SemaphoreType.DMA,
        ],
    )
    def _gather(cell_hbm, mask_hbm, table_hbm, out_hbm, idx_v, mask_v,
                words_v, sem0, sem1, msem):
        wid = lax.axis_index("s") * _NC + lax.axis_index("c")
        base = wid * _BPW
        pltpu.sync_copy(cell_hbm.at[pl.ds(base, _BPW)], idx_v)
        mask_cp = pltpu.async_copy(mask_hbm.at[pl.ds(base, _BPW)], mask_v,
                                   msem)
        sems = (sem0, sem1)

        def fire(c):
            lo = c * _CHUNK
            return pltpu.async_copy(
                table_hbm.at[idx_v.at[pl.ds(lo, _CHUNK)]],
                words_v.at[pl.ds(lo, _CHUNK)],
                sems[c % 2],
            )

        def mask_chunk(c):
            def body(i, carry):
                sl = pl.ds(c * _CHUNK + i * 16, 16)
                words_v[sl] = words_v[sl] & mask_v[sl]
                return carry

            lax.fori_loop(0, _CHUNK // 16, body, 0)

        pending = fire(0)
        mask_cp.wait()
        for c in range(1, _NCHUNK):
            nxt = fire(c)
            pending.wait()
            mask_chunk(c - 1)
            pending = nxt
        pending.wait()
        mask_chunk(_NCHUNK - 1)
        pltpu.sync_copy(words_v, out_hbm.at[pl.ds(base, _BPW)])

    return _gather


def kernel(norm_ray_origins, viewdirs, cache):
    o3 = norm_ray_origins.T.reshape(3, _R, _C)
    v3 = viewdirs.T.reshape(3, _R, _C)
    cell, mask = _compute_indices(o3, v3)
    table = _pack_table(cache.T).reshape(_NWORD)
    vals = _make_gather()(cell.reshape(_B), mask.reshape(_B), table)
    return vals.astype(jnp.bool_)


# speedup vs baseline: 6.9218x; 1.0241x over previous
import functools

import jax
import jax.numpy as jnp
from jax import lax
from jax.experimental import pallas as pl
from jax.experimental.pallas import tpu as pltpu
from jax.experimental.pallas import tpu_sc as plsc

_GRID_SIZE = 128
_MIDPOINT = 128
_B = 1048576
_NCELL = _GRID_SIZE ** 3
_NWORD = _NCELL // 4
_R = 1024
_C = 1024
_ROWS_PER_STEP = 128
_PACK_COLS = 131072

_NC = 2
_NS = 16
_NW = _NC * _NS
_BPW = _B // _NW
_CHUNK = 4096
_NCHUNK = _BPW // _CHUNK


def _spread_bits(x):
    x = x & jnp.uint32(0x3FF)
    x = (x | (x << 16)) & jnp.uint32(0x030000FF)
    x = (x | (x << 8)) & jnp.uint32(0x0300F00F)
    x = (x | (x << 4)) & jnp.uint32(0x030C30C3)
    x = (x | (x << 2)) & jnp.uint32(0x09249249)
    return x


def _pack_body(c0_ref, c1_ref, c2_ref, c3_ref, out_ref):
    def byte(ref):
        blk = ref[...]
        f = lax.broadcasted_iota(jnp.int32, blk.shape, 0)
        bits = jnp.where(blk > _MIDPOINT, jnp.int32(1) << f, jnp.int32(0))
        return jnp.sum(bits, axis=0, keepdims=True)

    word = (byte(c0_ref) | (byte(c1_ref) << 8) | (byte(c2_ref) << 16)
            | (byte(c3_ref) << 24))
    out_ref[...] = word[None]


def _pack_table(cache_t):
    grid = _NWORD // _PACK_COLS
    blocks_per_byte = _NWORD // _PACK_COLS

    def spec(k):
        return pl.BlockSpec((6, _PACK_COLS),
                            lambda j, k=k: (0, j + blocks_per_byte * k))

    return pl.pallas_call(
        _pack_body,
        grid=(grid,),
        in_specs=[spec(0), spec(1), spec(2), spec(3)],
        out_specs=pl.BlockSpec((1, 1, _PACK_COLS), lambda j: (j, 0, 0)),
        out_shape=jax.ShapeDtypeStruct((grid, 1, _PACK_COLS), jnp.int32),
    )(cache_t, cache_t, cache_t, cache_t)


def _index_body(o_ref, v_ref, cell_ref, mask_ref):
    vx = v_ref[0]
    vy = v_ref[1]
    vz = v_ref[2]
    denom = jnp.maximum(jnp.maximum(jnp.abs(vx), jnp.abs(vy)), jnp.abs(vz))
    a = vx / denom
    b = vy / denom
    c = vz / denom
    one = jnp.float32(1.0)
    conds = (a >= one, a <= -one, b >= one, b <= -one, c >= one, c <= -one)
    face = jnp.zeros(vx.shape, jnp.int32)
    for i, cond in enumerate(conds):
        face = jnp.where(cond, jnp.int32(i), face)

    def cell(p):
        q = jnp.clip((p / 2.0 + 0.5) * _GRID_SIZE, 0.0, float(_GRID_SIZE - 1))
        return _spread_bits(jnp.floor(q).astype(jnp.int32).astype(jnp.uint32))

    morton = (cell(o_ref[0]) | (cell(o_ref[1]) << 1)
              | (cell(o_ref[2]) << 2)).astype(jnp.int32)
    cell_ref[...] = morton & jnp.int32(_NWORD - 1)
    mask_ref[...] = jnp.int32(1) << (face + ((morton >> 19) << 3))


def _compute_indices(o3, v3, half):
    steps = _R // _ROWS_PER_STEP // 2
    off = half * steps
    return pl.pallas_call(
        _index_body,
        grid=(steps,),
        in_specs=[
            pl.BlockSpec((3, _ROWS_PER_STEP, _C), lambda i: (0, i + off, 0)),
            pl.BlockSpec((3, _ROWS_PER_STEP, _C), lambda i: (0, i + off, 0)),
        ],
        out_specs=[
            pl.BlockSpec((_ROWS_PER_STEP, _C), lambda i: (i, 0)),
            pl.BlockSpec((_ROWS_PER_STEP, _C), lambda i: (i, 0)),
        ],
        out_shape=[
            jax.ShapeDtypeStruct((_R // 2, _C), jnp.int32),
            jax.ShapeDtypeStruct((_R // 2, _C), jnp.int32),
        ],
    )(o3, v3)


@functools.cache
def _make_gather(n):
    bpw = n // _NW
    nchunk = bpw // _CHUNK

    @functools.partial(
        pl.kernel,
        mesh=plsc.VectorSubcoreMesh(core_axis_name="c", subcore_axis_name="s"),
        out_type=jax.ShapeDtypeStruct((n,), jnp.int32),
        scratch_types=[
            pltpu.VMEM((bpw,), jnp.int32),
            pltpu.VMEM((bpw,), jnp.int32),
            pltpu.VMEM((bpw,), jnp.int32),
            pltpu.SemaphoreType.DMA,
            pltpu.SemaphoreType.DMA,
            pltpu.SemaphoreType.DMA,
            pltpu.SemaphoreType.DMA,
        ],
    )
    def _gather(cell_hbm, mask_hbm, table_hbm, out_hbm, idx_v, mask_v,
                words_v, sem0, sem1, msem, osem):
        wid = lax.axis_index("s") * _NC + lax.axis_index("c")
        base = wid * bpw
        pltpu.sync_copy(cell_hbm.at[pl.ds(base, bpw)], idx_v)
        mask_cp = pltpu.async_copy(mask_hbm.at[pl.ds(base, bpw)], mask_v,
                                   msem)
        sems = (sem0, sem1)

        def fire(c):
            lo = c * _CHUNK
            return pltpu.async_copy(
                table_hbm.at[idx_v.at[pl.ds(lo, _CHUNK)]],
                words_v.at[pl.ds(lo, _CHUNK)],
                sems[c % 2],
            )

        def mask_chunk(c):
            def body(i, carry):
                sl = pl.ds(c * _CHUNK + i * 16, 16)
                words_v[sl] = words_v[sl] & mask_v[sl]
                return carry

            lax.fori_loop(0, _CHUNK // 16, body, 0)

        def store_chunk(c):
            lo = c * _CHUNK
            return pltpu.async_copy(
                words_v.at[pl.ds(lo, _CHUNK)],
                out_hbm.at[pl.ds(base + lo, _CHUNK)],
                osem,
            )

        out_cps = []
        pending = fire(0)
        mask_cp.wait()
        for c in range(1, nchunk):
            nxt = fire(c)
            pending.wait()
            mask_chunk(c - 1)
            out_cps.append(store_chunk(c - 1))
            pending = nxt
        pending.wait()
        mask_chunk(nchunk - 1)
        out_cps.append(store_chunk(nchunk - 1))
        for cp in out_cps:
            cp.wait()

    return _gather


def kernel(norm_ray_origins, viewdirs, cache):
    o3 = norm_ray_origins.T.reshape(3, _R, _C)
    v3 = viewdirs.T.reshape(3, _R, _C)
    half = _B // 2
    gather = _make_gather(half)
    table = _pack_table(cache.T).reshape(_NWORD)
    cell0, mask0 = _compute_indices(o3, v3, 0)
    vals0 = gather(cell0.reshape(half), mask0.reshape(half), table)
    cell1, mask1 = _compute_indices(o3, v3, 1)
    vals1 = gather(cell1.reshape(half), mask1.reshape(half), table)
    return jnp.concatenate(
        [vals0.astype(jnp.bool_), vals1.astype(jnp.bool_)])


# speedup vs baseline: 8.5411x; 1.2339x over previous
import functools

import jax
import jax.numpy as jnp
from jax import lax
from jax.experimental import pallas as pl
from jax.experimental.pallas import tpu as pltpu
from jax.experimental.pallas import tpu_sc as plsc

_GRID_SIZE = 128
_MIDPOINT = 128
_B = 1048576
_NCELL = _GRID_SIZE ** 3
_NWORD = _NCELL // 4
_R = 1024
_C = 1024
_ROWS_PER_STEP = 128
_PACK_COLS = 131072

_NC = 2
_NS = 16
_NW = _NC * _NS
_BPW = _B // _NW
_CHUNK = 4096
_NCHUNK = _BPW // _CHUNK


def _spread_bits(x):
    x = x & jnp.uint32(0x3FF)
    x = (x | (x << 16)) & jnp.uint32(0x030000FF)
    x = (x | (x << 8)) & jnp.uint32(0x0300F00F)
    x = (x | (x << 4)) & jnp.uint32(0x030C30C3)
    x = (x | (x << 2)) & jnp.uint32(0x09249249)
    return x


def _pack_body(c0_ref, c1_ref, c2_ref, c3_ref, out_ref):
    def byte(ref):
        blk = ref[...]
        f = lax.broadcasted_iota(jnp.int32, blk.shape, 0)
        bits = jnp.where(blk > _MIDPOINT, jnp.int32(1) << f, jnp.int32(0))
        return jnp.sum(bits, axis=0, keepdims=True)

    word = (byte(c0_ref) | (byte(c1_ref) << 8) | (byte(c2_ref) << 16)
            | (byte(c3_ref) << 24))
    out_ref[...] = word[None]


def _pack_table(cache_t):
    grid = _NWORD // _PACK_COLS
    blocks_per_byte = _NWORD // _PACK_COLS

    def spec(k):
        return pl.BlockSpec((6, _PACK_COLS),
                            lambda j, k=k: (0, j + blocks_per_byte * k))

    return pl.pallas_call(
        _pack_body,
        grid=(grid,),
        in_specs=[spec(0), spec(1), spec(2), spec(3)],
        out_specs=pl.BlockSpec((1, 1, _PACK_COLS), lambda j: (j, 0, 0)),
        out_shape=jax.ShapeDtypeStruct((grid, 1, _PACK_COLS), jnp.int32),
    )(cache_t, cache_t, cache_t, cache_t)


def _index_body(o_ref, v_ref, cell_ref, mask_ref):
    vx = v_ref[0]
    vy = v_ref[1]
    vz = v_ref[2]
    denom = jnp.maximum(jnp.maximum(jnp.abs(vx), jnp.abs(vy)), jnp.abs(vz))
    a = vx / denom
    b = vy / denom
    c = vz / denom
    one = jnp.float32(1.0)
    conds = (a >= one, a <= -one, b >= one, b <= -one, c >= one, c <= -one)
    face = jnp.zeros(vx.shape, jnp.int32)
    for i, cond in enumerate(conds):
        face = jnp.where(cond, jnp.int32(i), face)

    def cell(p):
        q = jnp.clip((p / 2.0 + 0.5) * _GRID_SIZE, 0.0, float(_GRID_SIZE - 1))
        return _spread_bits(jnp.floor(q).astype(jnp.int32).astype(jnp.uint32))

    morton = (cell(o_ref[0]) | (cell(o_ref[1]) << 1)
              | (cell(o_ref[2]) << 2)).astype(jnp.int32)
    cell_ref[...] = morton & jnp.int32(_NWORD - 1)
    mask_ref[...] = jnp.int32(1) << (face + ((morton >> 19) << 3))


def _compute_indices(o3, v3, half):
    steps = _R // _ROWS_PER_STEP // 2
    off = half * steps
    return pl.pallas_call(
        _index_body,
        grid=(steps,),
        in_specs=[
            pl.BlockSpec((3, _ROWS_PER_STEP, _C), lambda i: (0, i + off, 0)),
            pl.BlockSpec((3, _ROWS_PER_STEP, _C), lambda i: (0, i + off, 0)),
        ],
        out_specs=[
            pl.BlockSpec((_ROWS_PER_STEP, _C), lambda i: (i, 0)),
            pl.BlockSpec((_ROWS_PER_STEP, _C), lambda i: (i, 0)),
        ],
        out_shape=[
            jax.ShapeDtypeStruct((_R // 2, _C), jnp.int32),
            jax.ShapeDtypeStruct((_R // 2, _C), jnp.int32),
        ],
    )(o3, v3)


@functools.cache
def _make_gather(n):
    bpw = n // _NW
    nchunk = bpw // _CHUNK

    @functools.partial(
        pl.kernel,
        mesh=plsc.VectorSubcoreMesh(core_axis_name="c", subcore_axis_name="s"),
        out_type=jax.ShapeDtypeStruct((n,), jnp.int32),
        scratch_types=[
            pltpu.VMEM((bpw,), jnp.int32),
            pltpu.VMEM((bpw,), jnp.int32),
            pltpu.VMEM((bpw,), jnp.int32),
            pltpu.VMEM_SHARED((_NWORD,), jnp.int32),
            pltpu.SemaphoreType.DMA,
            pltpu.SemaphoreType.DMA,
            pltpu.SemaphoreType.DMA,
            pltpu.SemaphoreType.DMA,
        ],
    )
    def _gather(cell_hbm, mask_hbm, table_hbm, out_hbm, idx_v, mask_v,
                words_v, table_s, sem0, sem1, msem, osem):
        sid = lax.axis_index("s")
        wid = sid * _NC + lax.axis_index("c")
        base = wid * bpw
        stage = _NWORD // _NS
        pltpu.sync_copy(table_hbm.at[pl.ds(sid * stage, stage)],
                        table_s.at[pl.ds(sid * stage, stage)])
        pltpu.sync_copy(cell_hbm.at[pl.ds(base, bpw)], idx_v)
        mask_cp = pltpu.async_copy(mask_hbm.at[pl.ds(base, bpw)], mask_v,
                                   msem)
        plsc.subcore_barrier()
        sems = (sem0, sem1)

        def fire(c):
            lo = c * _CHUNK
            return pltpu.async_copy(
                table_s.at[idx_v.at[pl.ds(lo, _CHUNK)]],
                words_v.at[pl.ds(lo, _CHUNK)],
                sems[c % 2],
            )

        def mask_chunk(c):
            def body(i, carry):
                sl = pl.ds(c * _CHUNK + i * 16, 16)
                words_v[sl] = words_v[sl] & mask_v[sl]
                return carry

            lax.fori_loop(0, _CHUNK // 16, body, 0)

        def store_chunk(c):
            lo = c * _CHUNK
            return pltpu.async_copy(
                words_v.at[pl.ds(lo, _CHUNK)],
                out_hbm.at[pl.ds(base + lo, _CHUNK)],
                osem,
            )

        out_cps = []
        pending = fire(0)
        mask_cp.wait()
        for c in range(1, nchunk):
            nxt = fire(c)
            pending.wait()
            mask_chunk(c - 1)
            out_cps.append(store_chunk(c - 1))
            pending = nxt
        pending.wait()
        mask_chunk(nchunk - 1)
        out_cps.append(store_chunk(nchunk - 1))
        for cp in out_cps:
            cp.wait()

    return _gather


def kernel(norm_ray_origins, viewdirs, cache):
    o3 = norm_ray_origins.T.reshape(3, _R, _C)
    v3 = viewdirs.T.reshape(3, _R, _C)
    half = _B // 2
    gather = _make_gather(half)
    table = _pack_table(cache.T).reshape(_NWORD)
    cell0, mask0 = _compute_indices(o3, v3, 0)
    vals0 = gather(cell0.reshape(half), mask0.reshape(half), table)
    cell1, mask1 = _compute_indices(o3, v3, 1)
    vals1 = gather(cell1.reshape(half), mask1.reshape(half), table)
    return jnp.concatenate(
        [vals0.astype(jnp.bool_), vals1.astype(jnp.bool_)])


# speedup vs baseline: 9.3864x; 1.0990x over previous
import functools

import jax
import jax.numpy as jnp
from jax import lax
from jax.experimental import pallas as pl
from jax.experimental.pallas import tpu as pltpu
from jax.experimental.pallas import tpu_sc as plsc

_GRID_SIZE = 128
_MIDPOINT = 128
_B = 1048576
_NCELL = _GRID_SIZE ** 3
_NWORD = _NCELL // 4
_R = 1024
_C = 1024
_ROWS_PER_STEP = 128
_PACK_COLS = 131072

_NC = 2
_NS = 16
_NW = _NC * _NS
_BPW = _B // _NW
_CHUNK = 4096
_NCHUNK = _BPW // _CHUNK


def _spread_bits(x):
    x = x & jnp.uint32(0x3FF)
    x = (x | (x << 16)) & jnp.uint32(0x030000FF)
    x = (x | (x << 8)) & jnp.uint32(0x0300F00F)
    x = (x | (x << 4)) & jnp.uint32(0x030C30C3)
    x = (x | (x << 2)) & jnp.uint32(0x09249249)
    return x


def _pack_body(c0_ref, c1_ref, c2_ref, c3_ref, out_ref):
    def byte(ref):
        blk = ref[...]
        f = lax.broadcasted_iota(jnp.int32, blk.shape, 0)
        bits = jnp.where(blk > _MIDPOINT, jnp.int32(1) << f, jnp.int32(0))
        return jnp.sum(bits, axis=0, keepdims=True)

    word = (byte(c0_ref) | (byte(c1_ref) << 8) | (byte(c2_ref) << 16)
            | (byte(c3_ref) << 24))
    out_ref[...] = word[None]


def _pack_table(cache_t):
    grid = _NWORD // _PACK_COLS
    blocks_per_byte = _NWORD // _PACK_COLS

    def spec(k):
        return pl.BlockSpec((6, _PACK_COLS),
                            lambda j, k=k: (0, j + blocks_per_byte * k))

    return pl.pallas_call(
        _pack_body,
        grid=(grid,),
        in_specs=[spec(0), spec(1), spec(2), spec(3)],
        out_specs=pl.BlockSpec((1, 1, _PACK_COLS), lambda j: (j, 0, 0)),
        out_shape=jax.ShapeDtypeStruct((grid, 1, _PACK_COLS), jnp.int32),
    )(cache_t, cache_t, cache_t, cache_t)


def _index_body(o_ref, v_ref, cell_ref, mask_ref):
    vx = v_ref[0]
    vy = v_ref[1]
    vz = v_ref[2]
    denom = jnp.maximum(jnp.maximum(jnp.abs(vx), jnp.abs(vy)), jnp.abs(vz))
    a = vx / denom
    b = vy / denom
    c = vz / denom
    one = jnp.float32(1.0)
    conds = (a >= one, a <= -one, b >= one, b <= -one, c >= one, c <= -one)
    face = jnp.zeros(vx.shape, jnp.int32)
    for i, cond in enumerate(conds):
        face = jnp.where(cond, jnp.int32(i), face)

    def cell(p):
        q = jnp.clip((p / 2.0 + 0.5) * _GRID_SIZE, 0.0, float(_GRID_SIZE - 1))
        return _spread_bits(jnp.floor(q).astype(jnp.int32).astype(jnp.uint32))

    morton = (cell(o_ref[0]) | (cell(o_ref[1]) << 1)
              | (cell(o_ref[2]) << 2)).astype(jnp.int32)
    cell_ref[...] = morton & jnp.int32(_NWORD - 1)
    mask_ref[...] = jnp.int32(1) << (face + ((morton >> 19) << 3))


_IDX_ROWS = 512


def _compute_indices(o4, v4, half):
    rows = _B // 128 // 2
    steps = rows // _IDX_ROWS
    off = half * steps
    return pl.pallas_call(
        _index_body,
        grid=(steps,),
        in_specs=[
            pl.BlockSpec((3, _IDX_ROWS, 128), lambda i: (0, i + off, 0)),
            pl.BlockSpec((3, _IDX_ROWS, 128), lambda i: (0, i + off, 0)),
        ],
        out_specs=[
            pl.BlockSpec((_IDX_ROWS, 128), lambda i: (i, 0)),
            pl.BlockSpec((_IDX_ROWS, 128), lambda i: (i, 0)),
        ],
        out_shape=[
            jax.ShapeDtypeStruct((rows, 128), jnp.int32),
            jax.ShapeDtypeStruct((rows, 128), jnp.int32),
        ],
    )(o4, v4)


@functools.cache
def _make_gather(n):
    bpw = n // _NW
    nchunk = bpw // _CHUNK

    @functools.partial(
        pl.kernel,
        mesh=plsc.VectorSubcoreMesh(core_axis_name="c", subcore_axis_name="s"),
        out_type=jax.ShapeDtypeStruct((n,), jnp.int32),
        scratch_types=[
            pltpu.VMEM((bpw,), jnp.int32),
            pltpu.VMEM((bpw,), jnp.int32),
            pltpu.VMEM((bpw,), jnp.int32),
            pltpu.VMEM_SHARED((_NWORD,), jnp.int32),
            pltpu.SemaphoreType.DMA,
            pltpu.SemaphoreType.DMA,
            pltpu.SemaphoreType.DMA,
            pltpu.SemaphoreType.DMA,
        ],
    )
    def _gather(cell_hbm, mask_hbm, table_hbm, out_hbm, idx_v, mask_v,
                words_v, table_s, sem0, sem1, msem, osem):
        sid = lax.axis_index("s")
        wid = sid * _NC + lax.axis_index("c")
        base = wid * bpw
        stage = _NWORD // _NS
        pltpu.sync_copy(table_hbm.at[pl.ds(sid * stage, stage)],
                        table_s.at[pl.ds(sid * stage, stage)])
        pltpu.sync_copy(cell_hbm.at[pl.ds(base, bpw)], idx_v)
        mask_cp = pltpu.async_copy(mask_hbm.at[pl.ds(base, bpw)], mask_v,
                                   msem)
        plsc.subcore_barrier()
        sems = (sem0, sem1)

        def fire(c):
            lo = c * _CHUNK
            return pltpu.async_copy(
                table_s.at[idx_v.at[pl.ds(lo, _CHUNK)]],
                words_v.at[pl.ds(lo, _CHUNK)],
                sems[c % 2],
            )

        def mask_chunk(c):
            def body(i, carry):
                sl = pl.ds(c * _CHUNK + i * 16, 16)
                words_v[sl] = words_v[sl] & mask_v[sl]
                return carry

            lax.fori_loop(0, _CHUNK // 16, body, 0)

        def store_chunk(c):
            lo = c * _CHUNK
            return pltpu.async_copy(
                words_v.at[pl.ds(lo, _CHUNK)],
                out_hbm.at[pl.ds(base + lo, _CHUNK)],
                osem,
            )

        out_cps = []
        pending = fire(0)
        mask_cp.wait()
        for c in range(1, nchunk):
            nxt = fire(c)
            pending.wait()
            mask_chunk(c - 1)
            out_cps.append(store_chunk(c - 1))
            pending = nxt
        pending.wait()
        mask_chunk(nchunk - 1)
        out_cps.append(store_chunk(nchunk - 1))
        for cp in out_cps:
            cp.wait()

    return _gather


def kernel(norm_ray_origins, viewdirs, cache):
    o4 = norm_ray_origins.T.reshape(3, _B // 128, 128)
    v4 = viewdirs.T.reshape(3, _B // 128, 128)
    half = _B // 2
    gather = _make_gather(half)
    table = _pack_table(cache.T).reshape(_NWORD)
    cell0, mask0 = _compute_indices(o4, v4, 0)
    vals0 = gather(cell0.reshape(half), mask0.reshape(half), table)
    cell1, mask1 = _compute_indices(o4, v4, 1)
    vals1 = gather(cell1.reshape(half), mask1.reshape(half), table)
    return jnp.concatenate(
        [vals0.astype(jnp.bool_), vals1.astype(jnp.bool_)])


# speedup vs baseline: 9.5577x; 1.0182x over previous
import functools

import jax
import jax.numpy as jnp
from jax import lax
from jax.experimental import pallas as pl
from jax.experimental.pallas import tpu as pltpu
from jax.experimental.pallas import tpu_sc as plsc

_GRID_SIZE = 128
_MIDPOINT = 128
_B = 1048576
_NCELL = _GRID_SIZE ** 3
_NWORD = _NCELL // 4
_R = 1024
_C = 1024
_ROWS_PER_STEP = 128
_PACK_COLS = 131072

_NC = 2
_NS = 16
_NW = _NC * _NS
_BPW = _B // _NW
_CHUNK = 2048
_NCHUNK = _BPW // _CHUNK
_DEPTH = 3


def _spread_bits(x):
    x = x & jnp.uint32(0x3FF)
    x = (x | (x << 16)) & jnp.uint32(0x030000FF)
    x = (x | (x << 8)) & jnp.uint32(0x0300F00F)
    x = (x | (x << 4)) & jnp.uint32(0x030C30C3)
    x = (x | (x << 2)) & jnp.uint32(0x09249249)
    return x


def _pack_body(c0_ref, c1_ref, c2_ref, c3_ref, out_ref):
    def byte(ref):
        blk = ref[...]
        f = lax.broadcasted_iota(jnp.int32, blk.shape, 0)
        bits = jnp.where(blk > _MIDPOINT, jnp.int32(1) << f, jnp.int32(0))
        return jnp.sum(bits, axis=0, keepdims=True)

    word = (byte(c0_ref) | (byte(c1_ref) << 8) | (byte(c2_ref) << 16)
            | (byte(c3_ref) << 24))
    out_ref[...] = word[None]


def _pack_table(cache_t):
    grid = _NWORD // _PACK_COLS
    blocks_per_byte = _NWORD // _PACK_COLS

    def spec(k):
        return pl.BlockSpec((6, _PACK_COLS),
                            lambda j, k=k: (0, j + blocks_per_byte * k))

    return pl.pallas_call(
        _pack_body,
        grid=(grid,),
        in_specs=[spec(0), spec(1), spec(2), spec(3)],
        out_specs=pl.BlockSpec((1, 1, _PACK_COLS), lambda j: (j, 0, 0)),
        out_shape=jax.ShapeDtypeStruct((grid, 1, _PACK_COLS), jnp.int32),
    )(cache_t, cache_t, cache_t, cache_t)


def _index_body(o_ref, v_ref, cell_ref, mask_ref):
    vx = v_ref[0]
    vy = v_ref[1]
    vz = v_ref[2]
    denom = jnp.maximum(jnp.maximum(jnp.abs(vx), jnp.abs(vy)), jnp.abs(vz))
    a = vx / denom
    b = vy / denom
    c = vz / denom
    one = jnp.float32(1.0)
    conds = (a >= one, a <= -one, b >= one, b <= -one, c >= one, c <= -one)
    face = jnp.zeros(vx.shape, jnp.int32)
    for i, cond in enumerate(conds):
        face = jnp.where(cond, jnp.int32(i), face)

    def cell(p):
        q = jnp.clip((p / 2.0 + 0.5) * _GRID_SIZE, 0.0, float(_GRID_SIZE - 1))
        return _spread_bits(jnp.floor(q).astype(jnp.int32).astype(jnp.uint32))

    morton = (cell(o_ref[0]) | (cell(o_ref[1]) << 1)
              | (cell(o_ref[2]) << 2)).astype(jnp.int32)
    cell_ref[...] = morton & jnp.int32(_NWORD - 1)
    mask_ref[...] = jnp.int32(1) << (face + ((morton >> 19) << 3))


_IDX_ROWS = 512


def _compute_indices(o4, v4, half):
    rows = _B // 128 // 2
    steps = rows // _IDX_ROWS
    off = half * steps
    return pl.pallas_call(
        _index_body,
        grid=(steps,),
        in_specs=[
            pl.BlockSpec((3, _IDX_ROWS, 128), lambda i: (0, i + off, 0)),
            pl.BlockSpec((3, _IDX_ROWS, 128), lambda i: (0, i + off, 0)),
        ],
        out_specs=[
            pl.BlockSpec((_IDX_ROWS, 128), lambda i: (i, 0)),
            pl.BlockSpec((_IDX_ROWS, 128), lambda i: (i, 0)),
        ],
        out_shape=[
            jax.ShapeDtypeStruct((rows, 128), jnp.int32),
            jax.ShapeDtypeStruct((rows, 128), jnp.int32),
        ],
    )(o4, v4)


@functools.cache
def _make_gather(n):
    bpw = n // _NW
    nchunk = bpw // _CHUNK

    @functools.partial(
        pl.kernel,
        mesh=plsc.VectorSubcoreMesh(core_axis_name="c", subcore_axis_name="s"),
        out_type=jax.ShapeDtypeStruct((n,), jnp.int32),
        scratch_types=[
            pltpu.VMEM((bpw,), jnp.int32),
            pltpu.VMEM((bpw,), jnp.int32),
            pltpu.VMEM((bpw,), jnp.int32),
            pltpu.VMEM_SHARED((_NWORD,), jnp.int32),
            pltpu.SemaphoreType.DMA,
            pltpu.SemaphoreType.DMA,
            pltpu.SemaphoreType.DMA,
            pltpu.SemaphoreType.DMA,
            pltpu.SemaphoreType.DMA,
        ],
    )
    def _gather(cell_hbm, mask_hbm, table_hbm, out_hbm, idx_v, mask_v,
                words_v, table_s, sem0, sem1, sem2, msem, osem):
        sid = lax.axis_index("s")
        wid = sid * _NC + lax.axis_index("c")
        base = wid * bpw
        stage = _NWORD // _NS
        pltpu.sync_copy(table_hbm.at[pl.ds(sid * stage, stage)],
                        table_s.at[pl.ds(sid * stage, stage)])
        pltpu.sync_copy(cell_hbm.at[pl.ds(base, bpw)], idx_v)
        mask_cp = pltpu.async_copy(mask_hbm.at[pl.ds(base, bpw)], mask_v,
                                   msem)
        plsc.subcore_barrier()
        sems = (sem0, sem1, sem2)

        def fire(c):
            lo = c * _CHUNK
            return pltpu.async_copy(
                table_s.at[idx_v.at[pl.ds(lo, _CHUNK)]],
                words_v.at[pl.ds(lo, _CHUNK)],
                sems[c % _DEPTH],
            )

        def mask_chunk(c):
            def body(i, carry):
                sl = pl.ds(c * _CHUNK + i * 16, 16)
                words_v[sl] = words_v[sl] & mask_v[sl]
                return carry

            lax.fori_loop(0, _CHUNK // 16, body, 0)

        def store_chunk(c):
            lo = c * _CHUNK
            return pltpu.async_copy(
                words_v.at[pl.ds(lo, _CHUNK)],
                out_hbm.at[pl.ds(base + lo, _CHUNK)],
                osem,
            )

        out_cps = []
        ahead = _DEPTH - 1
        pend = [fire(c) for c in range(ahead)]
        mask_cp.wait()
        for c in range(nchunk):
            if c + ahead < nchunk:
                pend.append(fire(c + ahead))
            pend.pop(0).wait()
            mask_chunk(c)
            out_cps.append(store_chunk(c))
        for cp in out_cps:
            cp.wait()

    return _gather


def kernel(norm_ray_origins, viewdirs, cache):
    o4 = norm_ray_origins.T.reshape(3, _B // 128, 128)
    v4 = viewdirs.T.reshape(3, _B // 128, 128)
    half = _B // 2
    gather = _make_gather(half)
    table = _pack_table(cache.T).reshape(_NWORD)
    cell0, mask0 = _compute_indices(o4, v4, 0)
    vals0 = gather(cell0.reshape(half), mask0.reshape(half), table)
    cell1, mask1 = _compute_indices(o4, v4, 1)
    vals1 = gather(cell1.reshape(half), mask1.reshape(half), table)
    return jnp.concatenate(
        [vals0.astype(jnp.bool_), vals1.astype(jnp.bool_)])
